# baseline (device time: 164755 ns/iter reference)
import jax
import jax.numpy as jnp
from jax import lax
from jax.experimental import pallas as pl
from jax.experimental.pallas import tpu as pltpu

N_DEV = 4
SEG = 8


def kernel(x, w_mat):
    m, k_per = x.shape
    k_per2, n = w_mat.shape
    assert k_per == k_per2
    m_chunk = m // N_DEV
    m_half = m_chunk // 2
    m_seg = m_half // SEG
    n_hops = N_DEV - 1

    def body(x_ref, w_ref, out_ref, acc_r, acc_l, rcv_r, rcv_l,
             rs_ssem_r, rs_ssem_l, rs_rsem_r, rs_rsem_l,
             ag_ssem_r, ag_ssem_l, ag_rsem_r, ag_rsem_l):
        my_pos = lax.axis_index("i")
        left = (my_pos - 1) % N_DEV
        right = (my_pos + 1) % N_DEV

        def partial_rows(row0, nrows):
            xs = x_ref[pl.ds(row0, nrows), :]
            return jnp.dot(xs, w_ref[:, :], preferred_element_type=jnp.float32)

        def silu(y):
            return y * (1.0 / (1.0 + jnp.exp(-y)))

        def slab_row0(c, dirn):
            return c * m_chunk + dirn * m_half

        def rs_desc(dirn, h, s):
            acc, rcv = (acc_r, rcv_r) if dirn == 0 else (acc_l, rcv_l)
            ssem = rs_ssem_r if dirn == 0 else rs_ssem_l
            rsem = rs_rsem_r if dirn == 0 else rs_rsem_l
            tgt = right if dirn == 0 else left
            return pltpu.make_async_remote_copy(
                src_ref=acc.at[h, pl.ds(s * m_seg, m_seg), :],
                dst_ref=rcv.at[h, pl.ds(s * m_seg, m_seg), :],
                send_sem=ssem.at[h, s],
                recv_sem=rsem.at[h, s],
                device_id=(tgt,),
                device_id_type=pl.DeviceIdType.MESH,
            )

        def ag_desc(dirn, g, s):
            if dirn == 0:
                sc = (my_pos - g + 1) % N_DEV
                tgt = right
                ssem, rsem = ag_ssem_r, ag_rsem_r
            else:
                sc = (my_pos + g - 1) % N_DEV
                tgt = left
                ssem, rsem = ag_ssem_l, ag_rsem_l
            row0 = slab_row0(sc, dirn) + s * m_seg
            sl = out_ref.at[pl.ds(row0, m_seg), :]
            return pltpu.make_async_remote_copy(
                src_ref=sl, dst_ref=sl,
                send_sem=ssem.at[g, s],
                recv_sem=rsem.at[g, s],
                device_id=(tgt,),
                device_id_type=pl.DeviceIdType.MESH,
            )

        acc_r[0, :, :] = partial_rows(slab_row0(my_pos, 0), m_half)
        acc_l[0, :, :] = partial_rows(slab_row0(my_pos, 1), m_half)

        barrier_sem = pltpu.get_barrier_semaphore()
        pl.semaphore_signal(barrier_sem, inc=1, device_id=(left,),
                            device_id_type=pl.DeviceIdType.MESH)
        pl.semaphore_signal(barrier_sem, inc=1, device_id=(right,),
                            device_id_type=pl.DeviceIdType.MESH)
        pl.semaphore_wait(barrier_sem, 2)

        for s in range(SEG):
            rs_desc(0, 0, s).start()
            rs_desc(1, 0, s).start()
        for h in range(n_hops):
            cr = (my_pos - h - 1) % N_DEV
            cl = (my_pos + h + 1) % N_DEV
            for s in range(SEG):
                for dirn, c in ((0, cr), (1, cl)):
                    row0 = slab_row0(c, dirn) + s * m_seg
                    p = partial_rows(row0, m_seg)
                    rs_desc(dirn, h, s).wait_recv()
                    rcv = rcv_r if dirn == 0 else rcv_l
                    val = p + rcv[h, pl.ds(s * m_seg, m_seg), :]
                    if h < n_hops - 1:
                        acc = acc_r if dirn == 0 else acc_l
                        acc[h + 1, pl.ds(s * m_seg, m_seg), :] = val
                        rs_desc(dirn, h + 1, s).start()
                    else:
                        out_ref[pl.ds(row0, m_seg), :] = silu(val)
                        ag_desc(dirn, 0, s).start()

        for g in range(n_hops):
            for s in range(SEG):
                for dirn in (0, 1):
                    ag_desc(dirn, g, s).wait_recv()
                    if g < n_hops - 1:
                        ag_desc(dirn, g + 1, s).start()

        for h in range(n_hops):
            for s in range(SEG):
                for dirn in (0, 1):
                    rs_desc(dirn, h, s).wait_send()
                    ag_desc(dirn, h, s).wait_send()

    return pl.pallas_call(
        body,
        out_shape=jax.ShapeDtypeStruct((m, n), jnp.float32),
        in_specs=[
            pl.BlockSpec(memory_space=pltpu.VMEM),
            pl.BlockSpec(memory_space=pltpu.VMEM),
        ],
        out_specs=pl.BlockSpec(memory_space=pltpu.VMEM),
        scratch_shapes=[
            pltpu.VMEM((n_hops, m_half, n), jnp.float32),
            pltpu.VMEM((n_hops, m_half, n), jnp.float32),
            pltpu.VMEM((n_hops, m_half, n), jnp.float32),
            pltpu.VMEM((n_hops, m_half, n), jnp.float32),
            pltpu.SemaphoreType.DMA((n_hops, SEG)),
            pltpu.SemaphoreType.DMA((n_hops, SEG)),
            pltpu.SemaphoreType.DMA((n_hops, SEG)),
            pltpu.SemaphoreType.DMA((n_hops, SEG)),
            pltpu.SemaphoreType.DMA((n_hops, SEG)),
            pltpu.SemaphoreType.DMA((n_hops, SEG)),
            pltpu.SemaphoreType.DMA((n_hops, SEG)),
            pltpu.SemaphoreType.DMA((n_hops, SEG)),
        ],
        compiler_params=pltpu.CompilerParams(
            collective_id=0,
            vmem_limit_bytes=100 * 1024 * 1024,
        ),
    )(x, w_mat)


# device time: 163797 ns/iter; 1.0058x vs baseline; 1.0058x over previous
import jax
import jax.numpy as jnp
from jax import lax
from jax.experimental import pallas as pl
from jax.experimental.pallas import tpu as pltpu

N_DEV = 4
SEG = 4


def kernel(x, w_mat):
    m, k_per = x.shape
    k_per2, n = w_mat.shape
    assert k_per == k_per2
    m_chunk = m // N_DEV
    m_half = m_chunk // 2
    m_seg = m_half // SEG
    n_hops = N_DEV - 1

    def body(x_ref, w_ref, out_ref, acc_r, acc_l, rcv_r, rcv_l,
             rs_ssem_r, rs_ssem_l, rs_rsem_r, rs_rsem_l,
             ag_ssem_r, ag_ssem_l, ag_rsem_r, ag_rsem_l):
        my_pos = lax.axis_index("i")
        left = (my_pos - 1) % N_DEV
        right = (my_pos + 1) % N_DEV

        def partial_rows(row0, nrows):
            xs = x_ref[pl.ds(row0, nrows), :]
            return jnp.dot(xs, w_ref[:, :], preferred_element_type=jnp.float32)

        def silu(y):
            return y * (1.0 / (1.0 + jnp.exp(-y)))

        def slab_row0(c, dirn):
            return c * m_chunk + dirn * m_half

        def rs_desc(dirn, h, s):
            acc, rcv = (acc_r, rcv_r) if dirn == 0 else (acc_l, rcv_l)
            ssem = rs_ssem_r if dirn == 0 else rs_ssem_l
            rsem = rs_rsem_r if dirn == 0 else rs_rsem_l
            tgt = right if dirn == 0 else left
            return pltpu.make_async_remote_copy(
                src_ref=acc.at[h, pl.ds(s * m_seg, m_seg), :],
                dst_ref=rcv.at[h, pl.ds(s * m_seg, m_seg), :],
                send_sem=ssem.at[h, s],
                recv_sem=rsem.at[h, s],
                device_id=(tgt,),
                device_id_type=pl.DeviceIdType.MESH,
            )

        def ag_desc(dirn, g, s):
            if dirn == 0:
                sc = (my_pos - g + 1) % N_DEV
                tgt = right
                ssem, rsem = ag_ssem_r, ag_rsem_r
            else:
                sc = (my_pos + g - 1) % N_DEV
                tgt = left
                ssem, rsem = ag_ssem_l, ag_rsem_l
            row0 = slab_row0(sc, dirn) + s * m_seg
            sl = out_ref.at[pl.ds(row0, m_seg), :]
            return pltpu.make_async_remote_copy(
                src_ref=sl, dst_ref=sl,
                send_sem=ssem.at[g, s],
                recv_sem=rsem.at[g, s],
                device_id=(tgt,),
                device_id_type=pl.DeviceIdType.MESH,
            )

        acc_r[0, :, :] = partial_rows(slab_row0(my_pos, 0), m_half)
        acc_l[0, :, :] = partial_rows(slab_row0(my_pos, 1), m_half)

        barrier_sem = pltpu.get_barrier_semaphore()
        pl.semaphore_signal(barrier_sem, inc=1, device_id=(left,),
                            device_id_type=pl.DeviceIdType.MESH)
        pl.semaphore_signal(barrier_sem, inc=1, device_id=(right,),
                            device_id_type=pl.DeviceIdType.MESH)
        pl.semaphore_wait(barrier_sem, 2)

        for s in range(SEG):
            rs_desc(0, 0, s).start()
            rs_desc(1, 0, s).start()
        for h in range(n_hops):
            cr = (my_pos - h - 1) % N_DEV
            cl = (my_pos + h + 1) % N_DEV
            for s in range(SEG):
                for dirn, c in ((0, cr), (1, cl)):
                    row0 = slab_row0(c, dirn) + s * m_seg
                    p = partial_rows(row0, m_seg)
                    rs_desc(dirn, h, s).wait_recv()
                    rcv = rcv_r if dirn == 0 else rcv_l
                    val = p + rcv[h, pl.ds(s * m_seg, m_seg), :]
                    if h < n_hops - 1:
                        acc = acc_r if dirn == 0 else acc_l
                        acc[h + 1, pl.ds(s * m_seg, m_seg), :] = val
                        rs_desc(dirn, h + 1, s).start()
                    else:
                        out_ref[pl.ds(row0, m_seg), :] = silu(val)
                        ag_desc(dirn, 0, s).start()

        for g in range(n_hops):
            for s in range(SEG):
                for dirn in (0, 1):
                    ag_desc(dirn, g, s).wait_recv()
                    if g < n_hops - 1:
                        ag_desc(dirn, g + 1, s).start()

        for h in range(n_hops):
            for s in range(SEG):
                for dirn in (0, 1):
                    rs_desc(dirn, h, s).wait_send()
                    ag_desc(dirn, h, s).wait_send()

    return pl.pallas_call(
        body,
        out_shape=jax.ShapeDtypeStruct((m, n), jnp.float32),
        in_specs=[
            pl.BlockSpec(memory_space=pltpu.VMEM),
            pl.BlockSpec(memory_space=pltpu.VMEM),
        ],
        out_specs=pl.BlockSpec(memory_space=pltpu.VMEM),
        scratch_shapes=[
            pltpu.VMEM((n_hops, m_half, n), jnp.float32),
            pltpu.VMEM((n_hops, m_half, n), jnp.float32),
            pltpu.VMEM((n_hops, m_half, n), jnp.float32),
            pltpu.VMEM((n_hops, m_half, n), jnp.float32),
            pltpu.SemaphoreType.DMA((n_hops, SEG)),
            pltpu.SemaphoreType.DMA((n_hops, SEG)),
            pltpu.SemaphoreType.DMA((n_hops, SEG)),
            pltpu.SemaphoreType.DMA((n_hops, SEG)),
            pltpu.SemaphoreType.DMA((n_hops, SEG)),
            pltpu.SemaphoreType.DMA((n_hops, SEG)),
            pltpu.SemaphoreType.DMA((n_hops, SEG)),
            pltpu.SemaphoreType.DMA((n_hops, SEG)),
        ],
        compiler_params=pltpu.CompilerParams(
            collective_id=0,
            vmem_limit_bytes=100 * 1024 * 1024,
        ),
    )(x, w_mat)
